# baseline (speedup 1.0000x reference)
import functools

import jax
import jax.numpy as jnp
from jax import lax
from jax.experimental import pallas as pl
from jax.experimental.pallas import tpu as pltpu
from jax.experimental.pallas import tpu_sc as plsc

_D = 64
_DPAD = 128
_SCALE = 8.0

_NC = 2
_NS = 16
_NW = _NC * _NS

_NBUF = 2
_HALF = 2
_UNROLL = 4

_VBLK = 1024


def _prep_body(t_ref, o_ref):
    blk = t_ref[...].astype(jnp.float32)
    blk = blk.T * _SCALE
    o_ref[...] = jnp.pad(blk, ((0, 0), (0, _DPAD - _D)))


@functools.lru_cache(maxsize=None)
def _prep_fn(vocab):
    grid = (vocab + _VBLK - 1) // _VBLK
    return pl.pallas_call(
        _prep_body,
        grid=(grid,),
        in_specs=[pl.BlockSpec((_D, _VBLK), lambda i: (0, i))],
        out_specs=pl.BlockSpec((_VBLK, _DPAD), lambda i: (i, 0)),
        out_shape=jax.ShapeDtypeStruct((vocab, _DPAD), jnp.float32),
    )


@functools.lru_cache(maxsize=None)
def _emb_fn(n_tok, seq):
    tok_per_w = n_tok // _NW
    chunk = tok_per_w // _HALF
    n_steps = seq * _HALF
    mesh = plsc.VectorSubcoreMesh(core_axis_name="c", subcore_axis_name="s")

    scratch = [pltpu.VMEM((seq, tok_per_w), jnp.int32)]
    scratch += [pltpu.VMEM((chunk, _DPAD), jnp.float32) for _ in range(_NBUF)]
    scratch += [pltpu.VMEM((chunk, _D), jnp.float32) for _ in range(_NBUF)]
    scratch += [pltpu.SemaphoreType.DMA for _ in range(2 * _NBUF + 1)]

    @functools.partial(
        pl.kernel,
        mesh=mesh,
        compiler_params=pltpu.CompilerParams(use_tc_tiling_on_sc=False),
        out_type=jax.ShapeDtypeStruct((seq, n_tok, _D), jnp.float32),
        scratch_types=scratch,
    )
    def emb(table_hbm, xt_hbm, out_hbm, xbuf, *rest):
        gbufs = rest[:_NBUF]
        sbufs = rest[_NBUF:2 * _NBUF]
        gsem = rest[2 * _NBUF:3 * _NBUF]
        osem = rest[3 * _NBUF:4 * _NBUF]
        xsem = rest[4 * _NBUF]

        wid = lax.axis_index("s") * _NC + lax.axis_index("c")
        tok0 = wid * tok_per_w
        pltpu.async_copy(
            xt_hbm.at[:, pl.ds(tok0, tok_per_w)], xbuf, xsem
        ).wait()

        def compact_buf(gbuf, sbuf):
            def body(i, carry):
                r0 = i * _UNROLL
                for dr in range(_UNROLL):
                    for k in range(_D // 16):
                        sl = pl.ds(k * 16, 16)
                        sbuf[r0 + dr, sl] = gbuf[r0 + dr, sl]
                return carry

            lax.fori_loop(0, chunk // _UNROLL, body, 0)

        def pair_body(g, carry):
            for b in range(_NBUF):
                step = g * _NBUF + b
                s = step // _HALF
                h = step % _HALF
                dst = out_hbm.at[s, pl.ds(tok0 + h * chunk, chunk), :]

                @pl.when(g != 0)
                def _drain():
                    pltpu.make_async_copy(sbufs[b], dst, osem[b]).wait()

                pltpu.async_copy(
                    table_hbm.at[xbuf.at[s, pl.ds(h * chunk, chunk)]],
                    gbufs[b],
                    gsem[b],
                )
            for b in range(_NBUF):
                step = g * _NBUF + b
                s = step // _HALF
                h = step % _HALF
                dst = out_hbm.at[s, pl.ds(tok0 + h * chunk, chunk), :]
                pltpu.make_async_copy(
                    table_hbm.at[xbuf.at[s, pl.ds(h * chunk, chunk)]],
                    gbufs[b],
                    gsem[b],
                ).wait()
                compact_buf(gbufs[b], sbufs[b])
                pltpu.async_copy(sbufs[b], dst, osem[b])
            return carry

        lax.fori_loop(0, n_steps // _NBUF, pair_body, 0)
        for b in range(_NBUF):
            step = n_steps - _NBUF + b
            s = step // _HALF
            h = step % _HALF
            dst = out_hbm.at[s, pl.ds(tok0 + h * chunk, chunk), :]
            pltpu.make_async_copy(sbufs[b], dst, osem[b]).wait()

    return emb


def kernel(x, embedding_weight):
    n_tok, seq = x.shape
    vocab = embedding_weight.shape[0]
    table_p = _prep_fn(vocab)(embedding_weight.T)
    out_p = _emb_fn(n_tok, seq)(table_p, x.T)
    return out_p.transpose(1, 0, 2)


# speedup vs baseline: 1.1017x; 1.1017x over previous
import functools

import jax
import jax.numpy as jnp
from jax import lax
from jax.experimental import pallas as pl
from jax.experimental.pallas import tpu as pltpu
from jax.experimental.pallas import tpu_sc as plsc

_D = 64
_DPAD = 128
_SCALE = 8.0

_NC = 2
_NS = 16
_NW = _NC * _NS

_NBUF = 2
_VBLK = 1024


def _prep_body(t_ref, o_ref):
    blk = t_ref[...]
    eye = jnp.eye(_D, dtype=jnp.float32) * _SCALE
    blk_t = lax.dot_general(
        blk, eye, (((0,), (0,)), ((), ())),
        preferred_element_type=jnp.float32,
    )
    o_ref[...] = jnp.pad(blk_t, ((0, 0), (0, _DPAD - _D)))


@functools.lru_cache(maxsize=None)
def _prep_fn(vocab):
    grid = (vocab + _VBLK - 1) // _VBLK
    return pl.pallas_call(
        _prep_body,
        grid=(grid,),
        in_specs=[pl.BlockSpec((_D, _VBLK), lambda i: (0, i))],
        out_specs=pl.BlockSpec((_VBLK, _DPAD), lambda i: (i, 0)),
        out_shape=jax.ShapeDtypeStruct((vocab, _DPAD), jnp.float32),
    )


@functools.lru_cache(maxsize=None)
def _emb_fn(n_tok, seq):
    tok_per_w = n_tok // _NW
    mesh = plsc.VectorSubcoreMesh(core_axis_name="c", subcore_axis_name="s")

    scratch = [pltpu.VMEM((seq, tok_per_w), jnp.int32)]
    scratch += [pltpu.VMEM((tok_per_w, _D), jnp.float32) for _ in range(_NBUF)]
    scratch += [pltpu.SemaphoreType.DMA for _ in range(2 * _NBUF + 1)]

    @functools.partial(
        pl.kernel,
        mesh=mesh,
        compiler_params=pltpu.CompilerParams(use_tc_tiling_on_sc=False),
        out_type=jax.ShapeDtypeStruct((seq, n_tok, _D), jnp.float32),
        scratch_types=scratch,
    )
    def emb(table_hbm, xt_hbm, out_hbm, xbuf, *rest):
        bufs = rest[:_NBUF]
        gsem = rest[_NBUF:2 * _NBUF]
        osem = rest[2 * _NBUF:3 * _NBUF]
        xsem = rest[3 * _NBUF]

        wid = lax.axis_index("s") * _NC + lax.axis_index("c")
        tok0 = wid * tok_per_w
        pltpu.async_copy(
            xt_hbm.at[:, pl.ds(tok0, tok_per_w)], xbuf, xsem
        ).wait()

        def dbl_body(i, carry):
            for m in range(tok_per_w // 16):
                sl = pl.ds(m * 16, 16)
                xbuf[i, sl] = xbuf[i, sl] * 2
            return carry

        lax.fori_loop(0, seq, dbl_body, 0)

        def pair_body(g, carry):
            for b in range(_NBUF):
                s = g * _NBUF + b
                dst = out_hbm.at[s, pl.ds(tok0, tok_per_w), :]

                @pl.when(g != 0)
                def _drain():
                    pltpu.make_async_copy(bufs[b], dst, osem[b]).wait()

                pltpu.async_copy(
                    table_hbm.at[xbuf.at[s]], bufs[b], gsem[b]
                )
            for b in range(_NBUF):
                s = g * _NBUF + b
                dst = out_hbm.at[s, pl.ds(tok0, tok_per_w), :]
                pltpu.make_async_copy(
                    table_hbm.at[xbuf.at[s]], bufs[b], gsem[b]
                ).wait()
                pltpu.async_copy(bufs[b], dst, osem[b])
            return carry

        lax.fori_loop(0, seq // _NBUF, pair_body, 0)
        for b in range(_NBUF):
            s = seq - _NBUF + b
            dst = out_hbm.at[s, pl.ds(tok0, tok_per_w), :]
            pltpu.make_async_copy(bufs[b], dst, osem[b]).wait()

    return emb


def kernel(x, embedding_weight):
    n_tok, seq = x.shape
    vocab = embedding_weight.shape[0]
    table_p = _prep_fn(vocab)(embedding_weight.T)
    table_v = table_p.reshape(2 * vocab, _D)
    out_p = _emb_fn(n_tok, seq)(table_v, x.T)
    return out_p.transpose(1, 0, 2)


# speedup vs baseline: 1.8289x; 1.6601x over previous
import functools

import jax
import jax.numpy as jnp
from jax import lax
from jax.experimental import pallas as pl
from jax.experimental.pallas import tpu as pltpu
from jax.experimental.pallas import tpu_sc as plsc

_D = 64
_DPAD = 128
_SCALE = 8.0

_NC = 2
_NS = 16
_NW = _NC * _NS

_NBUF = 2
_VBLK = 4096


def _prep_body(t_ref, o_ref):
    blk = t_ref[...]
    eye = jnp.eye(_D, dtype=jnp.float32) * _SCALE
    blk_t = lax.dot_general(
        blk, eye, (((0,), (0,)), ((), ())),
        preferred_element_type=jnp.float32,
    )
    o_ref[...] = jnp.pad(blk_t, ((0, 0), (0, _DPAD - _D)))


@functools.lru_cache(maxsize=None)
def _prep_fn(vocab):
    grid = (vocab + _VBLK - 1) // _VBLK
    return pl.pallas_call(
        _prep_body,
        grid=(grid,),
        in_specs=[pl.BlockSpec((_D, _VBLK), lambda i: (0, i))],
        out_specs=pl.BlockSpec((_VBLK, _DPAD), lambda i: (i, 0)),
        out_shape=jax.ShapeDtypeStruct((vocab, _DPAD), jnp.float32),
    )


@functools.lru_cache(maxsize=None)
def _emb_fn(n_tok, seq):
    tok_per_w = n_tok // _NW
    mesh = plsc.VectorSubcoreMesh(core_axis_name="c", subcore_axis_name="s")

    scratch = [pltpu.VMEM((seq, tok_per_w), jnp.int32)]
    scratch += [pltpu.VMEM((tok_per_w, _D), jnp.float32) for _ in range(_NBUF)]
    scratch += [pltpu.SemaphoreType.DMA for _ in range(2 * _NBUF + 1)]

    @functools.partial(
        pl.kernel,
        mesh=mesh,
        compiler_params=pltpu.CompilerParams(use_tc_tiling_on_sc=False),
        out_type=jax.ShapeDtypeStruct((seq, n_tok, _D), jnp.float32),
        scratch_types=scratch,
    )
    def emb(table_hbm, xt_hbm, out_hbm, xbuf, *rest):
        bufs = rest[:_NBUF]
        gsem = rest[_NBUF:2 * _NBUF]
        osem = rest[2 * _NBUF:3 * _NBUF]
        xsem = rest[3 * _NBUF]

        wid = lax.axis_index("s") * _NC + lax.axis_index("c")
        tok0 = wid * tok_per_w
        pltpu.async_copy(
            xt_hbm.at[:, pl.ds(tok0, tok_per_w)], xbuf, xsem
        ).wait()

        def dbl_body(i, carry):
            for m in range(tok_per_w // 16):
                sl = pl.ds(m * 16, 16)
                xbuf[i, sl] = xbuf[i, sl] * 2
            return carry

        lax.fori_loop(0, seq, dbl_body, 0)

        def pair_body(g, carry):
            for b in range(_NBUF):
                s = g * _NBUF + b
                dst = out_hbm.at[s, pl.ds(tok0, tok_per_w), :]

                @pl.when(g != 0)
                def _drain():
                    pltpu.make_async_copy(bufs[b], dst, osem[b]).wait()

                pltpu.async_copy(
                    table_hbm.at[xbuf.at[s]], bufs[b], gsem[b]
                )
            for b in range(_NBUF):
                s = g * _NBUF + b
                dst = out_hbm.at[s, pl.ds(tok0, tok_per_w), :]
                pltpu.make_async_copy(
                    table_hbm.at[xbuf.at[s]], bufs[b], gsem[b]
                ).wait()
                pltpu.async_copy(bufs[b], dst, osem[b])
            return carry

        lax.fori_loop(0, seq // _NBUF, pair_body, 0)
        for b in range(_NBUF):
            s = seq - _NBUF + b
            dst = out_hbm.at[s, pl.ds(tok0, tok_per_w), :]
            pltpu.make_async_copy(bufs[b], dst, osem[b]).wait()

    return emb


def kernel(x, embedding_weight):
    n_tok, seq = x.shape
    vocab = embedding_weight.shape[0]
    table_p = _prep_fn(vocab)(embedding_weight.T)
    table_v = table_p.reshape(2 * vocab, _D)
    out_p = _emb_fn(n_tok, seq)(table_v, x.T)
    return out_p.transpose(1, 0, 2)


# speedup vs baseline: 2.0885x; 1.1419x over previous
import functools

import jax
import jax.numpy as jnp
from jax import lax
from jax.experimental import pallas as pl
from jax.experimental.pallas import tpu as pltpu
from jax.experimental.pallas import tpu_sc as plsc

_D = 64
_DPAD = 128
_SCALE = 8.0

_NC = 2
_NS = 16
_NW = _NC * _NS

_NBUF = 2
_VBLK = 8192


def _prep_body(t_ref, o_ref):
    blk = t_ref[...]
    eye = jnp.eye(_D, dtype=jnp.float32) * _SCALE
    blk_t = lax.dot_general(
        blk, eye, (((0,), (0,)), ((), ())),
        preferred_element_type=jnp.float32,
    )
    o_ref[...] = jnp.pad(blk_t, ((0, 0), (0, _DPAD - _D)))


@functools.lru_cache(maxsize=None)
def _prep_fn(vocab):
    grid = (vocab + _VBLK - 1) // _VBLK
    return pl.pallas_call(
        _prep_body,
        grid=(grid,),
        in_specs=[pl.BlockSpec((_D, _VBLK), lambda i: (0, i))],
        out_specs=pl.BlockSpec((_VBLK, _DPAD), lambda i: (i, 0)),
        out_shape=jax.ShapeDtypeStruct((vocab, _DPAD), jnp.float32),
    )


@functools.lru_cache(maxsize=None)
def _emb_fn(n_tok, seq):
    tok_per_w = n_tok // _NW
    mesh = plsc.VectorSubcoreMesh(core_axis_name="c", subcore_axis_name="s")

    scratch = [pltpu.VMEM((seq, tok_per_w), jnp.int32)]
    scratch += [pltpu.VMEM((tok_per_w, _D), jnp.float32) for _ in range(_NBUF)]
    scratch += [pltpu.SemaphoreType.DMA for _ in range(2 * _NBUF + 1)]

    @functools.partial(
        pl.kernel,
        mesh=mesh,
        compiler_params=pltpu.CompilerParams(use_tc_tiling_on_sc=False),
        out_type=jax.ShapeDtypeStruct((seq, n_tok, _D), jnp.float32),
        scratch_types=scratch,
    )
    def emb(table_hbm, xt_hbm, out_hbm, xbuf, *rest):
        bufs = rest[:_NBUF]
        gsem = rest[_NBUF:2 * _NBUF]
        osem = rest[2 * _NBUF:3 * _NBUF]
        xsem = rest[3 * _NBUF]

        wid = lax.axis_index("s") * _NC + lax.axis_index("c")
        tok0 = wid * tok_per_w
        pltpu.async_copy(
            xt_hbm.at[:, pl.ds(tok0, tok_per_w)], xbuf, xsem
        ).wait()

        def dbl_body(i, carry):
            for m in range(tok_per_w // 16):
                sl = pl.ds(m * 16, 16)
                xbuf[i, sl] = xbuf[i, sl] * 2
            return carry

        lax.fori_loop(0, seq, dbl_body, 0)

        def pair_body(g, carry):
            for b in range(_NBUF):
                s = g * _NBUF + b
                dst = out_hbm.at[s, pl.ds(tok0, tok_per_w), :]

                @pl.when(g != 0)
                def _drain():
                    pltpu.make_async_copy(bufs[b], dst, osem[b]).wait()

                pltpu.async_copy(
                    table_hbm.at[xbuf.at[s]], bufs[b], gsem[b]
                )
            for b in range(_NBUF):
                s = g * _NBUF + b
                dst = out_hbm.at[s, pl.ds(tok0, tok_per_w), :]
                pltpu.make_async_copy(
                    table_hbm.at[xbuf.at[s]], bufs[b], gsem[b]
                ).wait()
                pltpu.async_copy(bufs[b], dst, osem[b])
            return carry

        lax.fori_loop(0, seq // _NBUF, pair_body, 0)
        for b in range(_NBUF):
            s = seq - _NBUF + b
            dst = out_hbm.at[s, pl.ds(tok0, tok_per_w), :]
            pltpu.make_async_copy(bufs[b], dst, osem[b]).wait()

    return emb


def kernel(x, embedding_weight):
    n_tok, seq = x.shape
    vocab = embedding_weight.shape[0]
    table_p = _prep_fn(vocab)(embedding_weight.T)
    table_v = table_p.reshape(2 * vocab, _D)
    out_p = _emb_fn(n_tok, seq)(table_v, x.T)
    return out_p.transpose(1, 0, 2)


# speedup vs baseline: 2.2039x; 1.0553x over previous
import functools

import jax
import jax.numpy as jnp
from jax import lax
from jax.experimental import pallas as pl
from jax.experimental.pallas import tpu as pltpu
from jax.experimental.pallas import tpu_sc as plsc

_D = 64
_DPAD = 128
_SCALE = 8.0

_NC = 2
_NS = 16
_NW = _NC * _NS

_NBUF = 2
_VBLK = 16384


def _prep_body(t_ref, o_ref):
    blk = t_ref[...]
    eye = jnp.eye(_D, dtype=jnp.float32) * _SCALE
    blk_t = lax.dot_general(
        blk, eye, (((0,), (0,)), ((), ())),
        preferred_element_type=jnp.float32,
    )
    o_ref[...] = jnp.pad(blk_t, ((0, 0), (0, _DPAD - _D)))


@functools.lru_cache(maxsize=None)
def _prep_fn(vocab):
    grid = (vocab + _VBLK - 1) // _VBLK
    return pl.pallas_call(
        _prep_body,
        grid=(grid,),
        in_specs=[pl.BlockSpec((_D, _VBLK), lambda i: (0, i))],
        out_specs=pl.BlockSpec((_VBLK, _DPAD), lambda i: (i, 0)),
        out_shape=jax.ShapeDtypeStruct((vocab, _DPAD), jnp.float32),
    )


@functools.lru_cache(maxsize=None)
def _emb_fn(n_tok, seq):
    tok_per_w = n_tok // _NW
    mesh = plsc.VectorSubcoreMesh(core_axis_name="c", subcore_axis_name="s")

    scratch = [pltpu.VMEM((seq, tok_per_w), jnp.int32)]
    scratch += [pltpu.VMEM((tok_per_w, _D), jnp.float32) for _ in range(_NBUF)]
    scratch += [pltpu.SemaphoreType.DMA for _ in range(2 * _NBUF + 1)]

    @functools.partial(
        pl.kernel,
        mesh=mesh,
        compiler_params=pltpu.CompilerParams(use_tc_tiling_on_sc=False),
        out_type=jax.ShapeDtypeStruct((seq, n_tok, _D), jnp.float32),
        scratch_types=scratch,
    )
    def emb(table_hbm, xt_hbm, out_hbm, xbuf, *rest):
        bufs = rest[:_NBUF]
        gsem = rest[_NBUF:2 * _NBUF]
        osem = rest[2 * _NBUF:3 * _NBUF]
        xsem = rest[3 * _NBUF]

        wid = lax.axis_index("s") * _NC + lax.axis_index("c")
        tok0 = wid * tok_per_w
        pltpu.async_copy(
            xt_hbm.at[:, pl.ds(tok0, tok_per_w)], xbuf, xsem
        ).wait()

        def dbl_body(i, carry):
            for m in range(tok_per_w // 16):
                sl = pl.ds(m * 16, 16)
                xbuf[i, sl] = xbuf[i, sl] * 2
            return carry

        lax.fori_loop(0, seq, dbl_body, 0)

        def pair_body(g, carry):
            for b in range(_NBUF):
                s = g * _NBUF + b
                dst = out_hbm.at[s, pl.ds(tok0, tok_per_w), :]

                @pl.when(g != 0)
                def _drain():
                    pltpu.make_async_copy(bufs[b], dst, osem[b]).wait()

                pltpu.async_copy(
                    table_hbm.at[xbuf.at[s]], bufs[b], gsem[b]
                )
            for b in range(_NBUF):
                s = g * _NBUF + b
                dst = out_hbm.at[s, pl.ds(tok0, tok_per_w), :]
                pltpu.make_async_copy(
                    table_hbm.at[xbuf.at[s]], bufs[b], gsem[b]
                ).wait()
                pltpu.async_copy(bufs[b], dst, osem[b])
            return carry

        lax.fori_loop(0, seq // _NBUF, pair_body, 0)
        for b in range(_NBUF):
            s = seq - _NBUF + b
            dst = out_hbm.at[s, pl.ds(tok0, tok_per_w), :]
            pltpu.make_async_copy(bufs[b], dst, osem[b]).wait()

    return emb


def kernel(x, embedding_weight):
    n_tok, seq = x.shape
    vocab = embedding_weight.shape[0]
    table_p = _prep_fn(vocab)(embedding_weight.T)
    table_v = table_p.reshape(2 * vocab, _D)
    out_p = _emb_fn(n_tok, seq)(table_v, x.T)
    return out_p.transpose(1, 0, 2)


# speedup vs baseline: 2.2199x; 1.0072x over previous
import functools

import jax
import jax.numpy as jnp
from jax import lax
from jax.experimental import pallas as pl
from jax.experimental.pallas import tpu as pltpu
from jax.experimental.pallas import tpu_sc as plsc

_D = 64
_DPAD = 128
_SCALE = 8.0

_NC = 2
_NS = 16
_NW = _NC * _NS

_NBUF = 2
_VBLK = 32768


def _prep_body(t_ref, o_ref):
    blk = t_ref[...]
    eye = jnp.eye(_D, dtype=jnp.float32) * _SCALE
    blk_t = lax.dot_general(
        blk, eye, (((0,), (0,)), ((), ())),
        preferred_element_type=jnp.float32,
    )
    o_ref[...] = jnp.pad(blk_t, ((0, 0), (0, _DPAD - _D)))


@functools.lru_cache(maxsize=None)
def _prep_fn(vocab):
    grid = (vocab + _VBLK - 1) // _VBLK
    return pl.pallas_call(
        _prep_body,
        grid=(grid,),
        in_specs=[pl.BlockSpec((_D, _VBLK), lambda i: (0, i))],
        out_specs=pl.BlockSpec((_VBLK, _DPAD), lambda i: (i, 0)),
        out_shape=jax.ShapeDtypeStruct((vocab, _DPAD), jnp.float32),
    )


@functools.lru_cache(maxsize=None)
def _emb_fn(n_tok, seq):
    tok_per_w = n_tok // _NW
    mesh = plsc.VectorSubcoreMesh(core_axis_name="c", subcore_axis_name="s")

    scratch = [pltpu.VMEM((seq, tok_per_w), jnp.int32)]
    scratch += [pltpu.VMEM((tok_per_w, _D), jnp.float32) for _ in range(_NBUF)]
    scratch += [pltpu.SemaphoreType.DMA for _ in range(2 * _NBUF + 1)]

    @functools.partial(
        pl.kernel,
        mesh=mesh,
        compiler_params=pltpu.CompilerParams(use_tc_tiling_on_sc=False),
        out_type=jax.ShapeDtypeStruct((seq, n_tok, _D), jnp.float32),
        scratch_types=scratch,
    )
    def emb(table_hbm, xt_hbm, out_hbm, xbuf, *rest):
        bufs = rest[:_NBUF]
        gsem = rest[_NBUF:2 * _NBUF]
        osem = rest[2 * _NBUF:3 * _NBUF]
        xsem = rest[3 * _NBUF]

        wid = lax.axis_index("s") * _NC + lax.axis_index("c")
        tok0 = wid * tok_per_w
        pltpu.async_copy(
            xt_hbm.at[:, pl.ds(tok0, tok_per_w)], xbuf, xsem
        ).wait()

        def dbl_body(i, carry):
            for m in range(tok_per_w // 16):
                sl = pl.ds(m * 16, 16)
                xbuf[i, sl] = xbuf[i, sl] * 2
            return carry

        lax.fori_loop(0, seq, dbl_body, 0)

        def pair_body(g, carry):
            for b in range(_NBUF):
                s = g * _NBUF + b
                dst = out_hbm.at[s, pl.ds(tok0, tok_per_w), :]

                @pl.when(g != 0)
                def _drain():
                    pltpu.make_async_copy(bufs[b], dst, osem[b]).wait()

                pltpu.async_copy(
                    table_hbm.at[xbuf.at[s]], bufs[b], gsem[b]
                )
            for b in range(_NBUF):
                s = g * _NBUF + b
                dst = out_hbm.at[s, pl.ds(tok0, tok_per_w), :]
                pltpu.make_async_copy(
                    table_hbm.at[xbuf.at[s]], bufs[b], gsem[b]
                ).wait()
                pltpu.async_copy(bufs[b], dst, osem[b])
            return carry

        lax.fori_loop(0, seq // _NBUF, pair_body, 0)
        for b in range(_NBUF):
            s = seq - _NBUF + b
            dst = out_hbm.at[s, pl.ds(tok0, tok_per_w), :]
            pltpu.make_async_copy(bufs[b], dst, osem[b]).wait()

    return emb


def kernel(x, embedding_weight):
    n_tok, seq = x.shape
    vocab = embedding_weight.shape[0]
    table_p = _prep_fn(vocab)(embedding_weight.T)
    table_v = table_p.reshape(2 * vocab, _D)
    out_p = _emb_fn(n_tok, seq)(table_v, x.T)
    return out_p.transpose(1, 0, 2)
